# baseline (speedup 1.0000x reference)
import jax
import jax.numpy as jnp
from jax import lax
from jax.experimental import pallas as pl
from jax.experimental.pallas import tpu as pltpu

_C = 32
_GAMMA = 12.0
_CLIP = 0.01
_BN = 16
_S = 2048


def _lane_lt(width, period, shape):
    lane = lax.broadcasted_iota(jnp.int32, shape, 2)
    return (lane % period) < width


def _routing_body(x_ref, bw_ref, mask_ref, ranks_ref):
    sh = (_BN, _C, 128)
    lvl = []
    for j in range(0, _S // 128, 2):
        a = x_ref[:, :, 128 * j:128 * (j + 1)]
        b = x_ref[:, :, 128 * (j + 1):128 * (j + 2)]
        a = jnp.maximum(a, pltpu.roll(a, 128 - 16, 2))
        b = jnp.maximum(b, pltpu.roll(b, 128 - 16, 2))
        c = jnp.where(_lane_lt(16, 32, sh), a, pltpu.roll(b, 16, 2))
        lvl.append(c)
    w = 16
    while len(lvl) > 1:
        w //= 2
        nxt = []
        for j in range(0, len(lvl), 2):
            a = lvl[j]
            b = lvl[j + 1]
            a = jnp.maximum(a, pltpu.roll(a, 128 - w, 2))
            b = jnp.maximum(b, pltpu.roll(b, 128 - w, 2))
            nxt.append(jnp.where(_lane_lt(w, 2 * w, sh), a,
                                 pltpu.roll(b, w, 2)))
        lvl = nxt
    v = lvl[0]
    while w > 1:
        w //= 2
        v = jnp.maximum(v, pltpu.roll(v, 128 - w, 2))
    for r in (2, 4, 8, 16, 32, 64):
        v = v + pltpu.roll(v, 128 - r, 2)
    s = v[:, :, 0:1].reshape(_BN, _C)
    s = s * bw_ref[...]

    vk = s[:, :, None]
    vc = s[:, None, :]
    k_idx = lax.broadcasted_iota(jnp.int32, (_BN, _C, _C), 1)
    c_idx = lax.broadcasted_iota(jnp.int32, (_BN, _C, _C), 2)
    cmp = (vk > vc) | ((vk == vc) & (k_idx < c_idx))
    ranks = cmp.astype(jnp.int32).sum(axis=1)

    mask = jnp.exp(ranks.astype(jnp.float32) * (-_GAMMA / (_C - 1)))
    mask = jnp.where(mask < _CLIP, 0.0, mask)
    mask_ref[...] = mask
    ranks_ref[...] = ranks


def kernel(routings, boosting_weights):
    n = routings.shape[0]
    x = jnp.transpose(routings, (0, 2, 1))
    bw = boosting_weights.reshape(1, _C)
    mask, ranks = pl.pallas_call(
        _routing_body,
        grid=(n // _BN,),
        in_specs=[
            pl.BlockSpec((_BN, _C, _S), lambda i: (i, 0, 0)),
            pl.BlockSpec((1, _C), lambda i: (0, 0)),
        ],
        out_specs=[
            pl.BlockSpec((_BN, _C), lambda i: (i, 0)),
            pl.BlockSpec((_BN, _C), lambda i: (i, 0)),
        ],
        out_shape=[
            jax.ShapeDtypeStruct((n, _C), jnp.float32),
            jax.ShapeDtypeStruct((n, _C), jnp.int32),
        ],
    )(x, bw)
    return mask, ranks
